# Initial kernel scaffold; baseline (speedup 1.0000x reference)
import jax
import jax.numpy as jnp
from jax.experimental import pallas as pl


def kernel(ft0, ft1, pt0, pt1):
    raise NotImplementedError("write your pallas kernel here")


# speedup vs baseline: 2.7396x; 2.7396x over previous
import functools

import jax
import jax.numpy as jnp
from jax.experimental import pallas as pl
from jax.experimental.pallas import tpu as pltpu


def _sims_body(ft0_ref, ft1_ref, out_ref, *, scale):
    raw = jax.lax.dot_general(
        ft0_ref[...], ft1_ref[...],
        dimension_numbers=(((1,), (1,)), ((), ())),
        preferred_element_type=jnp.float32,
    )
    out_ref[...] = raw * scale + 0.5


def _post_body(sims_ref, pt0f_ref, pt0t_ref, pt1t_ref,
               simmax_ref, ratio_ref, cyc_ref, ixs_ref):
    sims = sims_ref[...]
    q, k = sims.shape

    rowmax = jnp.max(sims, axis=1, keepdims=True)
    colid = jax.lax.broadcasted_iota(jnp.int32, (q, k), 1)
    ixs = jnp.min(jnp.where(sims == rowmax, colid, k), axis=1,
                  keepdims=True)

    onehot = colid == ixs
    pt1x = pt1t_ref[0:1, :]
    pt1y = pt1t_ref[1:2, :]
    pt1mx = jnp.max(jnp.where(onehot, pt1x, -1.0), axis=1, keepdims=True)
    pt1my = jnp.max(jnp.where(onehot, pt1y, -1.0), axis=1, keepdims=True)

    near = (jnp.abs(pt1x - pt1mx) < 64.0) & (jnp.abs(pt1y - pt1my) < 64.0)
    sim_reverse = jnp.max(jnp.where(near, 0.0, sims), axis=1, keepdims=True)

    colmax = jnp.max(sims, axis=0, keepdims=True)
    rowid = jax.lax.broadcasted_iota(jnp.int32, (q, k), 0)
    colarg = jnp.min(jnp.where(sims == colmax, rowid, q), axis=0,
                     keepdims=True)
    ix_cyc = jnp.min(jnp.where(onehot, colarg, q), axis=1,
                     keepdims=True)

    qid = jax.lax.broadcasted_iota(jnp.int32, (q, q), 1)
    onehot2 = qid == ix_cyc
    pt0x = pt0t_ref[0:1, :]
    pt0y = pt0t_ref[1:2, :]
    pt0cx = jnp.max(jnp.where(onehot2, pt0x, -1.0), axis=1, keepdims=True)
    pt0cy = jnp.max(jnp.where(onehot2, pt0y, -1.0), axis=1, keepdims=True)
    dx = pt0cx - pt0f_ref[:, 0:1]
    dy = pt0cy - pt0f_ref[:, 1:2]

    simmax_ref[...] = rowmax
    ratio_ref[...] = rowmax / sim_reverse
    cyc_ref[...] = jnp.sqrt(dx * dx + dy * dy)
    ixs_ref[...] = ixs


def kernel(ft0, ft1, pt0, pt1):
    nq, c, h, w = ft0.shape
    nk = ft1.shape[0]
    d = c * h * w
    scale = 1.0 / (h ** 2) / 2.0

    ft0f = ft0.reshape(nq, d)
    ft1f = ft1.reshape(nk, d)

    nblk = 512 if nk % 512 == 0 else nk
    sims = pl.pallas_call(
        functools.partial(_sims_body, scale=scale),
        grid=(nk // nblk,),
        in_specs=[
            pl.BlockSpec((nq, d), lambda i: (0, 0)),
            pl.BlockSpec((nblk, d), lambda i: (i, 0)),
        ],
        out_specs=pl.BlockSpec((nq, nblk), lambda i: (0, i)),
        out_shape=jax.ShapeDtypeStruct((nq, nk), jnp.float32),
    )(ft0f, ft1f)

    pt0f = pt0.astype(jnp.float32)
    pt0t = pt0f.T
    pt1t = pt1.astype(jnp.float32).T

    simmax, ratios, cyc, ixs = pl.pallas_call(
        _post_body,
        out_shape=(
            jax.ShapeDtypeStruct((nq, 1), jnp.float32),
            jax.ShapeDtypeStruct((nq, 1), jnp.float32),
            jax.ShapeDtypeStruct((nq, 1), jnp.float32),
            jax.ShapeDtypeStruct((nq, 1), jnp.int32),
        ),
    )(sims, pt0f, pt0t, pt1t)

    return (simmax.reshape(nq), ratios.reshape(nq),
            cyc.reshape(nq), ixs.reshape(nq))
